# Initial kernel scaffold; baseline (speedup 1.0000x reference)
import jax
import jax.numpy as jnp
from jax.experimental import pallas as pl


def kernel(input, action_embedding):
    raise NotImplementedError("write your pallas kernel here")


# speedup vs baseline: 2.9747x; 2.9747x over previous
import functools

import jax
import jax.numpy as jnp
from jax import lax
from jax.experimental import pallas as pl
from jax.experimental.pallas import tpu as pltpu
from jax.experimental.pallas import tpu_sc as plsc

_NUM_CORES = 2
_NUM_SUBCORES = 16
_NUM_WORKERS = _NUM_CORES * _NUM_SUBCORES
_CHUNK = 128


def _sc_gather(idx_hbm, table_hbm, out_hbm, idx_v, rows_v, gsem):
    wid = lax.axis_index("s") * _NUM_CORES + lax.axis_index("c")
    n_chunks = idx_v.shape[0]
    pltpu.sync_copy(idx_hbm.at[wid], idx_v)

    def body(j, carry):
        pltpu.async_copy(table_hbm.at[idx_v.at[j]], rows_v, gsem).wait()
        pltpu.sync_copy(rows_v, out_hbm.at[wid, j])
        return carry

    lax.fori_loop(0, n_chunks, body, 0)


def kernel(input, action_embedding):
    n_rows = input.size
    dim = action_embedding.shape[1]
    idx_flat = input.reshape(-1).astype(jnp.int32)

    slab = _NUM_WORKERS * _CHUNK
    pad = (-n_rows) % slab
    if pad:
        idx_flat = jnp.pad(idx_flat, (0, pad))
    n_chunks = idx_flat.size // slab
    idx3 = idx_flat.reshape(_NUM_WORKERS, n_chunks, _CHUNK)

    mesh = plsc.VectorSubcoreMesh(core_axis_name="c", subcore_axis_name="s")
    run = functools.partial(
        pl.kernel,
        mesh=mesh,
        out_type=jax.ShapeDtypeStruct(
            (_NUM_WORKERS, n_chunks, _CHUNK, dim), jnp.float32
        ),
        scratch_types=[
            pltpu.VMEM((n_chunks, _CHUNK), jnp.int32),
            pltpu.VMEM((_CHUNK, dim), jnp.float32),
            pltpu.SemaphoreType.DMA,
        ],
    )(_sc_gather)

    out = run(idx3, action_embedding)
    out = out.reshape(-1, dim)[:n_rows]
    return out.reshape(*input.shape, dim)


# speedup vs baseline: 3.3434x; 1.1240x over previous
import functools

import jax
import jax.numpy as jnp
from jax import lax
from jax.experimental import pallas as pl
from jax.experimental.pallas import tpu as pltpu
from jax.experimental.pallas import tpu_sc as plsc

_NUM_CORES = 2
_NUM_SUBCORES = 16
_NUM_WORKERS = _NUM_CORES * _NUM_SUBCORES
_CHUNK = 128


def _sc_gather(idx_hbm, table_hbm, out_hbm, idx_v, rows0, rows1, gsem0,
               gsem1, osem0, osem1):
    wid = lax.axis_index("s") * _NUM_CORES + lax.axis_index("c")
    n_chunks = idx_v.shape[0]
    pltpu.sync_copy(idx_hbm.at[wid], idx_v)

    pltpu.async_copy(table_hbm.at[idx_v.at[0]], rows0, gsem0)
    pltpu.async_copy(table_hbm.at[idx_v.at[1]], rows1, gsem1)

    def body(k, carry):
        j0 = 2 * k
        pltpu.make_async_copy(rows0, out_hbm.at[wid, j0], gsem0).wait()
        pltpu.async_copy(rows0, out_hbm.at[wid, j0], osem0)

        @pl.when(j0 + 2 < n_chunks)
        def _():
            pltpu.make_async_copy(rows0, out_hbm.at[wid, j0], osem0).wait()
            pltpu.async_copy(table_hbm.at[idx_v.at[j0 + 2]], rows0, gsem0)

        pltpu.make_async_copy(rows1, out_hbm.at[wid, j0 + 1], gsem1).wait()
        pltpu.async_copy(rows1, out_hbm.at[wid, j0 + 1], osem1)

        @pl.when(j0 + 3 < n_chunks)
        def _():
            pltpu.make_async_copy(rows1, out_hbm.at[wid, j0 + 1],
                                  osem1).wait()
            pltpu.async_copy(table_hbm.at[idx_v.at[j0 + 3]], rows1, gsem1)

        return carry

    lax.fori_loop(0, n_chunks // 2, body, 0)

    pltpu.make_async_copy(rows0, out_hbm.at[wid, n_chunks - 2], osem0).wait()
    pltpu.make_async_copy(rows1, out_hbm.at[wid, n_chunks - 1], osem1).wait()


def kernel(input, action_embedding):
    n_rows = input.size
    dim = action_embedding.shape[1]
    idx_flat = input.reshape(-1).astype(jnp.int32)

    slab = _NUM_WORKERS * _CHUNK * 2
    pad = (-n_rows) % slab
    if pad:
        idx_flat = jnp.pad(idx_flat, (0, pad))
    n_chunks = idx_flat.size // (_NUM_WORKERS * _CHUNK)
    idx3 = idx_flat.reshape(_NUM_WORKERS, n_chunks, _CHUNK)

    mesh = plsc.VectorSubcoreMesh(core_axis_name="c", subcore_axis_name="s")
    run = functools.partial(
        pl.kernel,
        mesh=mesh,
        out_type=jax.ShapeDtypeStruct(
            (_NUM_WORKERS, n_chunks, _CHUNK, dim), jnp.float32
        ),
        scratch_types=[
            pltpu.VMEM((n_chunks, _CHUNK), jnp.int32),
            pltpu.VMEM((_CHUNK, dim), jnp.float32),
            pltpu.VMEM((_CHUNK, dim), jnp.float32),
            pltpu.SemaphoreType.DMA,
            pltpu.SemaphoreType.DMA,
            pltpu.SemaphoreType.DMA,
            pltpu.SemaphoreType.DMA,
        ],
    )(_sc_gather)

    out = run(idx3, action_embedding)
    out = out.reshape(-1, dim)[:n_rows]
    return out.reshape(*input.shape, dim)


# speedup vs baseline: 5.1293x; 1.5342x over previous
import functools

import jax
import jax.numpy as jnp
from jax import lax
from jax.experimental import pallas as pl
from jax.experimental.pallas import tpu as pltpu
from jax.experimental.pallas import tpu_sc as plsc

_NUM_CORES = 2
_NUM_SUBCORES = 16
_NUM_WORKERS = _NUM_CORES * _NUM_SUBCORES


def _sc_gather(idx_hbm, table_hbm, out_hbm, idx_v, rows0, rows1, gsem0,
               gsem1, osem0, osem1):
    wid = lax.axis_index("s") * _NUM_CORES + lax.axis_index("c")
    n_chunks = idx_v.shape[0]
    pltpu.sync_copy(idx_hbm.at[wid], idx_v)

    def fire_gather(j, rows, gsem):
        pltpu.async_copy(table_hbm.at[idx_v.at[j]], rows, gsem)

    def wait_gather(rows, gsem):
        pltpu.make_async_copy(table_hbm.at[idx_v.at[0]], rows, gsem).wait()

    def fire_out(j, rows, osem):
        pltpu.async_copy(rows, out_hbm.at[wid * n_chunks + j], osem)

    def wait_out(rows, osem):
        pltpu.make_async_copy(rows, out_hbm.at[0], osem).wait()

    fire_gather(0, rows0, gsem0)
    fire_gather(1, rows1, gsem1)

    def body(k, carry):
        j0 = 2 * k
        wait_gather(rows0, gsem0)
        fire_out(j0, rows0, osem0)

        @pl.when(j0 + 2 < n_chunks)
        def _():
            wait_out(rows0, osem0)
            fire_gather(j0 + 2, rows0, gsem0)

        wait_gather(rows1, gsem1)
        fire_out(j0 + 1, rows1, osem1)

        @pl.when(j0 + 3 < n_chunks)
        def _():
            wait_out(rows1, osem1)
            fire_gather(j0 + 3, rows1, gsem1)

        return carry

    lax.fori_loop(0, n_chunks // 2, body, 0)

    wait_out(rows0, osem0)
    wait_out(rows1, osem1)


def kernel(input, action_embedding):
    batch, hist = input.shape
    dim = action_embedding.shape[1]

    idx = input.astype(jnp.int32)
    bslab = _NUM_WORKERS * 2
    bpad = (-batch) % bslab
    if bpad:
        idx = jnp.pad(idx, ((0, bpad), (0, 0)))
    n_chunks = idx.shape[0] // _NUM_WORKERS
    idx3 = idx.reshape(_NUM_WORKERS, n_chunks, hist)

    mesh = plsc.VectorSubcoreMesh(core_axis_name="c", subcore_axis_name="s")
    run = functools.partial(
        pl.kernel,
        mesh=mesh,
        out_type=jax.ShapeDtypeStruct((idx.shape[0], hist, dim),
                                      jnp.float32),
        scratch_types=[
            pltpu.VMEM((n_chunks, hist), jnp.int32),
            pltpu.VMEM((hist, dim), jnp.float32),
            pltpu.VMEM((hist, dim), jnp.float32),
            pltpu.SemaphoreType.DMA,
            pltpu.SemaphoreType.DMA,
            pltpu.SemaphoreType.DMA,
            pltpu.SemaphoreType.DMA,
        ],
    )(_sc_gather)

    out = run(idx3, action_embedding)
    if bpad:
        out = out[:batch]
    return out


# speedup vs baseline: 10.3617x; 2.0201x over previous
import functools

import jax
import jax.numpy as jnp
from jax import lax
from jax.experimental import pallas as pl
from jax.experimental.pallas import tpu as pltpu
from jax.experimental.pallas import tpu_sc as plsc

_NUM_CORES = 2
_NUM_SUBCORES = 16
_NUM_WORKERS = _NUM_CORES * _NUM_SUBCORES
_CHUNK = 128


def _sc_gather(idx_hbm, table_hbm, out_hbm, idx_v, rows0, rows1, gsem0,
               gsem1, osem0, osem1):
    wid = lax.axis_index("s") * _NUM_CORES + lax.axis_index("c")
    n_chunks = idx_v.shape[0] // _CHUNK
    base = wid * idx_v.shape[0]
    pltpu.sync_copy(idx_hbm.at[pl.ds(base, idx_v.shape[0])], idx_v)

    def fire_gather(j, rows, gsem):
        pltpu.async_copy(table_hbm.at[idx_v.at[pl.ds(j * _CHUNK, _CHUNK)]],
                         rows, gsem)

    def wait_gather(rows, gsem):
        pltpu.make_async_copy(table_hbm.at[idx_v.at[pl.ds(0, _CHUNK)]],
                              rows, gsem).wait()

    def fire_out(j, rows, osem):
        pltpu.async_copy(rows, out_hbm.at[pl.ds(base + j * _CHUNK, _CHUNK)],
                         osem)

    def wait_out(rows, osem):
        pltpu.make_async_copy(rows, out_hbm.at[pl.ds(0, _CHUNK)],
                              osem).wait()

    fire_gather(0, rows0, gsem0)
    fire_gather(1, rows1, gsem1)

    def body(k, carry):
        j0 = 2 * k
        wait_gather(rows0, gsem0)
        fire_out(j0, rows0, osem0)

        @pl.when(j0 + 2 < n_chunks)
        def _():
            wait_out(rows0, osem0)
            fire_gather(j0 + 2, rows0, gsem0)

        wait_gather(rows1, gsem1)
        fire_out(j0 + 1, rows1, osem1)

        @pl.when(j0 + 3 < n_chunks)
        def _():
            wait_out(rows1, osem1)
            fire_gather(j0 + 3, rows1, gsem1)

        return carry

    lax.fori_loop(0, n_chunks // 2, body, 0)

    wait_out(rows0, osem0)
    wait_out(rows1, osem1)


def kernel(input, action_embedding):
    batch, hist = input.shape
    dim = action_embedding.shape[1]
    n_rows = batch * hist

    idx_flat = jnp.transpose(input).astype(jnp.int32).reshape(-1)
    slab = _NUM_WORKERS * _CHUNK * 2
    pad = (-n_rows) % slab
    if pad:
        idx_flat = jnp.pad(idx_flat, (0, pad))
    rows_per_worker = idx_flat.size // _NUM_WORKERS

    mesh = plsc.VectorSubcoreMesh(core_axis_name="c", subcore_axis_name="s")
    run = functools.partial(
        pl.kernel,
        mesh=mesh,
        out_type=jax.ShapeDtypeStruct((idx_flat.size, dim), jnp.float32),
        scratch_types=[
            pltpu.VMEM((rows_per_worker,), jnp.int32),
            pltpu.VMEM((_CHUNK, dim), jnp.float32),
            pltpu.VMEM((_CHUNK, dim), jnp.float32),
            pltpu.SemaphoreType.DMA,
            pltpu.SemaphoreType.DMA,
            pltpu.SemaphoreType.DMA,
            pltpu.SemaphoreType.DMA,
        ],
    )(_sc_gather)

    out = run(idx_flat, action_embedding)
    out = out[:n_rows].reshape(hist, batch, dim)
    return jnp.transpose(out, (1, 0, 2))


# speedup vs baseline: 10.3972x; 1.0034x over previous
import functools

import jax
import jax.numpy as jnp
from jax import lax
from jax.experimental import pallas as pl
from jax.experimental.pallas import tpu as pltpu
from jax.experimental.pallas import tpu_sc as plsc

_NUM_CORES = 2
_NUM_SUBCORES = 16
_NUM_WORKERS = _NUM_CORES * _NUM_SUBCORES
_CHUNK = 128
_NBUF = 3


def _sc_gather(idx_hbm, table_hbm, out_hbm, idx_v, rows0, rows1, rows2,
               gsem0, gsem1, gsem2, osem0, osem1, osem2):
    wid = lax.axis_index("s") * _NUM_CORES + lax.axis_index("c")
    n_chunks = idx_v.shape[0] // _CHUNK
    base = wid * idx_v.shape[0]
    pltpu.sync_copy(idx_hbm.at[pl.ds(base, idx_v.shape[0])], idx_v)

    bufs = ((rows0, gsem0, osem0), (rows1, gsem1, osem1),
            (rows2, gsem2, osem2))

    def fire_gather(j, rows, gsem):
        pltpu.async_copy(table_hbm.at[idx_v.at[pl.ds(j * _CHUNK, _CHUNK)]],
                         rows, gsem)

    def wait_gather(rows, gsem):
        pltpu.make_async_copy(table_hbm.at[idx_v.at[pl.ds(0, _CHUNK)]],
                              rows, gsem).wait()

    def fire_out(j, rows, osem):
        pltpu.async_copy(rows, out_hbm.at[pl.ds(base + j * _CHUNK, _CHUNK)],
                         osem)

    def wait_out(rows, osem):
        pltpu.make_async_copy(rows, out_hbm.at[pl.ds(0, _CHUNK)],
                              osem).wait()

    for b in range(_NBUF):
        fire_gather(b, bufs[b][0], bufs[b][1])

    def body(k, carry):
        j0 = _NBUF * k
        for b in range(_NBUF):
            rows, gsem, osem = bufs[b]
            j = j0 + b

            @pl.when(j < n_chunks)
            def _():
                wait_gather(rows, gsem)
                fire_out(j, rows, osem)

                @pl.when(j + _NBUF < n_chunks)
                def _():
                    wait_out(rows, osem)
                    fire_gather(j + _NBUF, rows, gsem)

        return carry

    lax.fori_loop(0, -(-n_chunks // _NBUF), body, 0)

    for b in range(_NBUF):
        wait_out(bufs[b][0], bufs[b][2])


def kernel(input, action_embedding):
    batch, hist = input.shape
    dim = action_embedding.shape[1]
    n_rows = batch * hist

    idx_flat = jnp.transpose(input).astype(jnp.int32).reshape(-1)
    slab = _NUM_WORKERS * _CHUNK
    pad = (-n_rows) % slab
    if pad:
        idx_flat = jnp.pad(idx_flat, (0, pad))
    rows_per_worker = idx_flat.size // _NUM_WORKERS

    mesh = plsc.VectorSubcoreMesh(core_axis_name="c", subcore_axis_name="s")
    run = functools.partial(
        pl.kernel,
        mesh=mesh,
        out_type=jax.ShapeDtypeStruct((idx_flat.size, dim), jnp.float32),
        scratch_types=[
            pltpu.VMEM((rows_per_worker,), jnp.int32),
            pltpu.VMEM((_CHUNK, dim), jnp.float32),
            pltpu.VMEM((_CHUNK, dim), jnp.float32),
            pltpu.VMEM((_CHUNK, dim), jnp.float32),
            pltpu.SemaphoreType.DMA,
            pltpu.SemaphoreType.DMA,
            pltpu.SemaphoreType.DMA,
            pltpu.SemaphoreType.DMA,
            pltpu.SemaphoreType.DMA,
            pltpu.SemaphoreType.DMA,
        ],
    )(_sc_gather)

    out = run(idx_flat, action_embedding)
    out = out[:n_rows].reshape(hist, batch, dim)
    return jnp.transpose(out, (1, 0, 2))


# speedup vs baseline: 10.4090x; 1.0011x over previous
import functools

import jax
import jax.numpy as jnp
from jax import lax
from jax.experimental import pallas as pl
from jax.experimental.pallas import tpu as pltpu
from jax.experimental.pallas import tpu_sc as plsc

_NUM_CORES = 2
_NUM_SUBCORES = 16
_NUM_WORKERS = _NUM_CORES * _NUM_SUBCORES
_CHUNK = 128
_GPB = 2
_SUPER = _CHUNK * _GPB
_NBUF = 3


def _sc_gather(idx_hbm, table_hbm, out_hbm, idx_v, rows0, rows1, rows2,
               gsem0, gsem1, gsem2, osem0, osem1, osem2):
    wid = lax.axis_index("s") * _NUM_CORES + lax.axis_index("c")
    n_chunks = idx_v.shape[0] // _SUPER
    base = wid * idx_v.shape[0]
    pltpu.sync_copy(idx_hbm.at[pl.ds(base, idx_v.shape[0])], idx_v)

    bufs = ((rows0, gsem0, osem0), (rows1, gsem1, osem1),
            (rows2, gsem2, osem2))

    def fire_gather(j, rows, gsem):
        for i in range(_GPB):
            pltpu.async_copy(
                table_hbm.at[idx_v.at[pl.ds(j * _SUPER + i * _CHUNK,
                                            _CHUNK)]],
                rows.at[pl.ds(i * _CHUNK, _CHUNK)], gsem)

    def wait_gather(rows, gsem):
        for i in range(_GPB):
            pltpu.make_async_copy(table_hbm.at[idx_v.at[pl.ds(0, _CHUNK)]],
                                  rows.at[pl.ds(i * _CHUNK, _CHUNK)],
                                  gsem).wait()

    def fire_out(j, rows, osem):
        pltpu.async_copy(rows, out_hbm.at[pl.ds(base + j * _SUPER, _SUPER)],
                         osem)

    def wait_out(rows, osem):
        pltpu.make_async_copy(rows, out_hbm.at[pl.ds(0, _SUPER)],
                              osem).wait()

    for b in range(_NBUF):
        fire_gather(b, bufs[b][0], bufs[b][1])

    def body(k, carry):
        j0 = _NBUF * k
        for b in range(_NBUF):
            rows, gsem, osem = bufs[b]
            j = j0 + b

            @pl.when(j < n_chunks)
            def _():
                wait_gather(rows, gsem)
                fire_out(j, rows, osem)

                @pl.when(j + _NBUF < n_chunks)
                def _():
                    wait_out(rows, osem)
                    fire_gather(j + _NBUF, rows, gsem)

        return carry

    lax.fori_loop(0, -(-n_chunks // _NBUF), body, 0)

    for b in range(_NBUF):
        wait_out(bufs[b][0], bufs[b][2])


def kernel(input, action_embedding):
    batch, hist = input.shape
    dim = action_embedding.shape[1]
    n_rows = batch * hist

    idx_flat = jnp.transpose(input).astype(jnp.int32).reshape(-1)
    slab = _NUM_WORKERS * _SUPER
    pad = (-n_rows) % slab
    if pad:
        idx_flat = jnp.pad(idx_flat, (0, pad))
    rows_per_worker = idx_flat.size // _NUM_WORKERS

    mesh = plsc.VectorSubcoreMesh(core_axis_name="c", subcore_axis_name="s")
    run = functools.partial(
        pl.kernel,
        mesh=mesh,
        out_type=jax.ShapeDtypeStruct((idx_flat.size, dim), jnp.float32),
        scratch_types=[
            pltpu.VMEM((rows_per_worker,), jnp.int32),
            pltpu.VMEM((_SUPER, dim), jnp.float32),
            pltpu.VMEM((_SUPER, dim), jnp.float32),
            pltpu.VMEM((_SUPER, dim), jnp.float32),
            pltpu.SemaphoreType.DMA,
            pltpu.SemaphoreType.DMA,
            pltpu.SemaphoreType.DMA,
            pltpu.SemaphoreType.DMA,
            pltpu.SemaphoreType.DMA,
            pltpu.SemaphoreType.DMA,
        ],
    )(_sc_gather)

    out = run(idx_flat, action_embedding)
    out = out[:n_rows].reshape(hist, batch, dim)
    return jnp.transpose(out, (1, 0, 2))
